# baseline (speedup 1.0000x reference)
import functools

import jax
import jax.numpy as jnp
from jax import lax
from jax.experimental import pallas as pl
from jax.experimental.pallas import tpu as pltpu
from jax.experimental.pallas import tpu_sc as plsc

_NC = 2
_NS = 16
_NW = _NC * _NS
_L = 16
_C = 16
_H = 768
_NV = _H // _L


def _rsqrt_sc(xv):
    iv = plsc.bitcast(xv, jnp.int32)
    y = plsc.bitcast(
        jnp.full((_L,), 0x5F3759DF, jnp.int32) - (iv >> 1), jnp.float32)
    hx = xv * 0.5
    y = y * (1.5 - hx * y * y)
    y = y * (1.5 - hx * y * y)
    y = y * (1.5 - hx * y * y)
    return y


def _sc_body(inp, pos, gam, bet, out,
             in_v, pos_v, t_v, g_v, b_v, sem_ld, sem_out):
    B, S, H = inp.shape
    wid = lax.axis_index("s") * _NC + lax.axis_index("c")
    s_per_w = S // _NW
    nchunk = s_per_w // _C
    base = wid * s_per_w
    rows = B * _C
    pltpu.sync_copy(gam, g_v)
    pltpu.sync_copy(bet, b_v)
    lane = jnp.arange(_L, dtype=jnp.int32)
    perms = [lane ^ (1 << k) for k in range(4)]

    def ld_copies(ci):
        half = ci % 2
        cs = [pltpu.make_async_copy(
            pos.at[pl.ds(base + ci * _C, _C)],
            pos_v.at[pl.ds(half * _C, _C)], sem_ld)]
        for b in range(B):
            cs.append(pltpu.make_async_copy(
                inp.at[b, pl.ds(base + ci * _C, _C)],
                in_v.at[pl.ds(half * rows + b * _C, _C)], sem_ld))
        return cs

    def out_copies(ci):
        half = ci % 2
        return [pltpu.make_async_copy(
            in_v.at[pl.ds(half * rows + b * _C, _C)],
            out.at[b, pl.ds(base + ci * _C, _C)], sem_out)
            for b in range(B)]

    for c in ld_copies(0):
        c.start()

    def chunk_body(ci, _):
        half = ci % 2
        poff = half * _C
        boff = half * rows
        for c in ld_copies(ci):
            c.wait()

        def quad_body(i, _):
            r0 = i * 4
            br = boff + r0
            pr = poff + (r0 & (_C - 1))
            a0 = [jnp.zeros((_L,), jnp.float32) for _ in range(8)]
            a2 = [jnp.zeros((_L,), jnp.float32) for _ in range(8)]
            for j in range(_NV):
                js = pl.ds(j * _L, _L)
                k2 = j % 2
                for k in range(4):
                    t = in_v[br + k, js] + pos_v[pr + k, js]
                    t_v[k, js] = t
                    a0[k2 * 4 + k] = a0[k2 * 4 + k] + t
                    a2[k2 * 4 + k] = a2[k2 * 4 + k] + t * t
            stats = []
            for k in range(4):
                acc = a0[k] + a0[4 + k]
                sq = a2[k] + a2[4 + k]
                for p16 in perms:
                    acc = acc + jnp.take_along_axis(acc, p16, axis=0)
                    sq = sq + jnp.take_along_axis(sq, p16, axis=0)
                m = acc * (1.0 / _H)
                y = _rsqrt_sc(sq * (1.0 / _H) - m * m + 1e-5)
                stats.append((y, m * y))
            for j in range(_NV):
                js = pl.ds(j * _L, _L)
                g = g_v[js]
                bb = b_v[js]
                for k in range(4):
                    y, o = stats[k]
                    in_v[br + k, js] = (t_v[k, js] * y - o) * g + bb
            return 0

        lax.fori_loop(0, rows // 4, quad_body, 0)

        @pl.when(ci > 0)
        def _():
            for c in out_copies(ci - 1):
                c.wait()

        for c in out_copies(ci):
            c.start()

        @pl.when(ci + 1 < nchunk)
        def _():
            for c in ld_copies(ci + 1):
                c.start()

        return 0

    lax.fori_loop(0, nchunk, chunk_body, 0)
    for c in out_copies(nchunk - 1):
        c.wait()


def _sc_layernorm(input_feat, pos_table, ln_gamma, ln_beta):
    B, S, H = input_feat.shape
    mesh = plsc.VectorSubcoreMesh(core_axis_name="c", subcore_axis_name="s")
    fn = pl.kernel(
        _sc_body,
        mesh=mesh,
        compiler_params=pltpu.CompilerParams(
            use_tc_tiling_on_sc=False, needs_layout_passes=False),
        out_type=jax.ShapeDtypeStruct((B, S, H), jnp.float32),
        scratch_types=[
            pltpu.VMEM((2 * B * _C, H), jnp.float32),
            pltpu.VMEM((2 * _C, H), jnp.float32),
            pltpu.VMEM((4, H), jnp.float32),
            pltpu.VMEM((H,), jnp.float32),
            pltpu.VMEM((H,), jnp.float32),
            pltpu.SemaphoreType.DMA,
            pltpu.SemaphoreType.DMA,
        ],
    )
    return fn(input_feat, pos_table, ln_gamma, ln_beta)


def kernel(input_feat, pos_table, ln_gamma, ln_beta):
    return _sc_layernorm(input_feat, pos_table, ln_gamma, ln_beta)


# speedup vs baseline: 2.2556x; 2.2556x over previous
import functools

import jax
import jax.numpy as jnp
from jax import lax
from jax.experimental import pallas as pl
from jax.experimental.pallas import tpu as pltpu
from jax.experimental.pallas import tpu_sc as plsc

_NC = 2
_NS = 16
_NW = _NC * _NS
_L = 16
_C = 16
_H = 768
_NV = _H // _L


def _rsqrt_sc(xv):
    iv = plsc.bitcast(xv, jnp.int32)
    y = plsc.bitcast(
        jnp.full((_L,), 0x5F3759DF, jnp.int32) - (iv >> 1), jnp.float32)
    hx = xv * 0.5
    y = y * (1.5 - hx * y * y)
    y = y * (1.5 - hx * y * y)
    y = y * (1.5 - hx * y * y)
    return y


def _sc_body(inp, pos, gam, bet, out,
             pos_v, in0, in1, in2, in3, g_v, b_v,
             sem_pos, sem_i0, sem_i1, sem_i2, sem_i3,
             sem_o0, sem_o1, sem_o2, sem_o3):
    B, S, H = inp.shape
    in_v = (in0, in1, in2, in3)
    sem_in = (sem_i0, sem_i1, sem_i2, sem_i3)
    sem_out = (sem_o0, sem_o1, sem_o2, sem_o3)
    wid = lax.axis_index("s") * _NC + lax.axis_index("c")
    s_per_w = S // _NW
    nchunk = s_per_w // _C
    base = wid * s_per_w
    pltpu.sync_copy(gam, g_v)
    pltpu.sync_copy(bet, b_v)
    lane = jnp.arange(_L, dtype=jnp.int32)
    perms = [lane ^ (1 << k) for k in range(4)]

    def pos_src(ci):
        return pos.at[pl.ds(base + ci * _C, _C)]

    def pos_dst(ci):
        return pos_v.at[pl.ds((ci % 2) * _C, _C)]

    def in_src(b, ci):
        return inp.at[b, pl.ds(base + ci * _C, _C)]

    def out_dst(b, ci):
        return out.at[b, pl.ds(base + ci * _C, _C)]

    pltpu.async_copy(pos_src(0), pos_dst(0), sem_pos)
    for b in range(B):
        pltpu.async_copy(in_src(b, 0), in_v[b], sem_in[b])

    def compute_slab(b, poff):
        buf = in_v[b]

        def pair_body(i, _):
            r0 = i * 2
            a0 = [jnp.zeros((_L,), jnp.float32) for _ in range(8)]
            a2 = [jnp.zeros((_L,), jnp.float32) for _ in range(8)]
            for j in range(_NV):
                js = pl.ds(j * _L, _L)
                t0 = buf[r0, js] + pos_v[poff + r0, js]
                t1 = buf[r0 + 1, js] + pos_v[poff + r0 + 1, js]
                k = j % 4
                a0[k] = a0[k] + t0
                a2[k] = a2[k] + t0 * t0
                a0[4 + k] = a0[4 + k] + t1
                a2[4 + k] = a2[4 + k] + t1 * t1
            acc0 = (a0[0] + a0[1]) + (a0[2] + a0[3])
            acc1 = (a0[4] + a0[5]) + (a0[6] + a0[7])
            sq0 = (a2[0] + a2[1]) + (a2[2] + a2[3])
            sq1 = (a2[4] + a2[5]) + (a2[6] + a2[7])
            for p16 in perms:
                acc0 = acc0 + jnp.take_along_axis(acc0, p16, axis=0)
                acc1 = acc1 + jnp.take_along_axis(acc1, p16, axis=0)
                sq0 = sq0 + jnp.take_along_axis(sq0, p16, axis=0)
                sq1 = sq1 + jnp.take_along_axis(sq1, p16, axis=0)
            m0 = acc0 * (1.0 / _H)
            m1 = acc1 * (1.0 / _H)
            y0 = _rsqrt_sc(sq0 * (1.0 / _H) - m0 * m0 + 1e-5)
            y1 = _rsqrt_sc(sq1 * (1.0 / _H) - m1 * m1 + 1e-5)
            o0 = m0 * y0
            o1 = m1 * y1
            for j in range(_NV):
                js = pl.ds(j * _L, _L)
                g = g_v[js]
                bb = b_v[js]
                t0 = buf[r0, js] + pos_v[poff + r0, js]
                t1 = buf[r0 + 1, js] + pos_v[poff + r0 + 1, js]
                buf[r0, js] = (t0 * y0 - o0) * g + bb
                buf[r0 + 1, js] = (t1 * y1 - o1) * g + bb
            return 0

        lax.fori_loop(0, _C // 2, pair_body, 0)

    def chunk_body(ci, _):
        poff = (ci % 2) * _C
        @pl.when(ci > 0)
        def _():
            pltpu.make_async_copy(in_v[3], out_dst(3, ci - 1), sem_out[3]).wait()
            pltpu.async_copy(in_src(3, ci), in_v[3], sem_in[3])

        pltpu.make_async_copy(pos_src(ci), pos_dst(ci), sem_pos).wait()

        @pl.when(ci + 1 < nchunk)
        def _():
            pltpu.async_copy(pos_src(ci + 1), pos_dst(ci + 1), sem_pos)

        for b in range(B):
            pltpu.make_async_copy(in_src(b, ci), in_v[b], sem_in[b]).wait()
            compute_slab(b, poff)
            pltpu.async_copy(in_v[b], out_dst(b, ci), sem_out[b])
            if b >= 1:
                pb = b - 1

                @pl.when(ci + 1 < nchunk)
                def _():
                    pltpu.make_async_copy(
                        in_v[pb], out_dst(pb, ci), sem_out[pb]).wait()
                    pltpu.async_copy(in_src(pb, ci + 1), in_v[pb], sem_in[pb])
        return 0

    lax.fori_loop(0, nchunk, chunk_body, 0)
    last = nchunk - 1
    for b in range(B - 1):
        pltpu.make_async_copy(in_v[b], out_dst(b, last), sem_out[b]).wait()
    pltpu.make_async_copy(in_v[3], out_dst(3, last), sem_out[3]).wait()


def _sc_layernorm(input_feat, pos_table, ln_gamma, ln_beta):
    B, S, H = input_feat.shape
    mesh = plsc.VectorSubcoreMesh(core_axis_name="c", subcore_axis_name="s")
    fn = pl.kernel(
        _sc_body,
        mesh=mesh,
        compiler_params=pltpu.CompilerParams(
            use_tc_tiling_on_sc=False, needs_layout_passes=False),
        out_type=jax.ShapeDtypeStruct((B, S, H), jnp.float32),
        scratch_types=[
            pltpu.VMEM((2 * _C, H), jnp.float32),
            pltpu.VMEM((_C, H), jnp.float32),
            pltpu.VMEM((_C, H), jnp.float32),
            pltpu.VMEM((_C, H), jnp.float32),
            pltpu.VMEM((_C, H), jnp.float32),
            pltpu.VMEM((H,), jnp.float32),
            pltpu.VMEM((H,), jnp.float32),
        ] + [pltpu.SemaphoreType.DMA] * 9,
    )
    return fn(input_feat, pos_table, ln_gamma, ln_beta)


def kernel(input_feat, pos_table, ln_gamma, ln_beta):
    return _sc_layernorm(input_feat, pos_table, ln_gamma, ln_beta)
